# baseline (speedup 1.0000x reference)
import functools

import jax
import jax.numpy as jnp
from jax import lax
from jax.experimental import pallas as pl
from jax.experimental.pallas import tpu as pltpu
from jax.experimental.pallas import tpu_sc as plsc

M = 50000
D = 256
DW = D // 2
K = 16
L = 16

NC, NS = 2, 16
NW = NC * NS
C_OUT = 8
G_ROWS = C_OUT * K

CH0 = 260
CH1 = 132
TOT_CHUNKS = NS * (CH0 + CH1)
M_PAD = TOT_CHUNKS * C_OUT
IDX_ROWS = NS * CH0 + (NS - 1) * CH1 + CH0


def _sc_max_pool(features_bf, pools_chunked):
    mesh = plsc.VectorSubcoreMesh(core_axis_name="c", subcore_axis_name="s")

    @functools.partial(
        pl.kernel,
        mesh=mesh,
        out_type=jax.ShapeDtypeStruct((M_PAD, DW), jnp.uint32),
        scratch_types=[
            pltpu.VMEM((CH0, 1, G_ROWS), jnp.int32),
            pltpu.VMEM((G_ROWS, DW), jnp.uint32),
            pltpu.VMEM((G_ROWS, DW), jnp.uint32),
            pltpu.VMEM((C_OUT, DW), jnp.uint32),
            pltpu.VMEM((C_OUT, DW), jnp.uint32),
            pltpu.SemaphoreType.DMA,
            pltpu.SemaphoreType.DMA,
            pltpu.SemaphoreType.DMA,
        ],
    )
    def kern(feat_hbm, idx_hbm, out_hbm, idx_all, rows0, rows1, outb0,
             outb1, sem0, sem1, sem_o):
        cidx = lax.axis_index("c")
        sidx = lax.axis_index("s")
        is0 = cidx == 0
        my_ch = lax.select(is0, jnp.int32(CH0), jnp.int32(CH1))
        row0_of_tile = lax.select(
            is0, sidx * CH0, NS * CH0 + sidx * CH1)
        base_w = row0_of_tile * C_OUT

        pltpu.sync_copy(idx_hbm.at[pl.ds(row0_of_tile, CH0)], idx_all)

        def fire(g, rows, sem):
            pltpu.async_copy(feat_hbm.at[idx_all.at[g, 0]], rows, sem)

        def wait_gather(g, rows, sem):
            pltpu.make_async_copy(
                feat_hbm.at[idx_all.at[g, 0]], rows, sem).wait()

        def reduce_chunk(rows, outb):
            def r_body(r, c):
                row0 = r * K
                for j in range(DW // L):
                    col = j * L
                    bc = lax.bitcast_convert_type
                    x = rows[row0, pl.ds(col, L)]
                    lo = bc(x << 16, jnp.float32)
                    hi = bc(x & jnp.uint32(0xFFFF0000), jnp.float32)
                    for k in range(1, K):
                        x = rows[row0 + k, pl.ds(col, L)]
                        lo = jnp.maximum(lo, bc(x << 16, jnp.float32))
                        hi = jnp.maximum(
                            hi, bc(x & jnp.uint32(0xFFFF0000), jnp.float32))
                    outb[r, pl.ds(col, L)] = (
                        (bc(lo, jnp.uint32) >> 16) | bc(hi, jnp.uint32))
                return c

            lax.fori_loop(0, C_OUT, r_body, 0)

        def out_copy(g, outb):
            pltpu.async_copy(
                outb, out_hbm.at[pl.ds(base_w + g * C_OUT, C_OUT)], sem_o)

        def drain_one_out(outb):
            pltpu.make_async_copy(
                outb, out_hbm.at[pl.ds(base_w, C_OUT)], sem_o).wait()

        fire(0, rows0, sem0)
        fire(1, rows1, sem1)

        def pair_body(t, c):
            g0 = 2 * t
            wait_gather(g0, rows0, sem0)

            @pl.when(t > 0)
            def _():
                drain_one_out(outb0)

            reduce_chunk(rows0, outb0)
            out_copy(g0, outb0)
            fire(g0 + 2, rows0, sem0)

            wait_gather(g0 + 1, rows1, sem1)

            @pl.when(t > 0)
            def _():
                drain_one_out(outb1)

            reduce_chunk(rows1, outb1)
            out_copy(g0 + 1, outb1)
            fire(g0 + 3, rows1, sem1)
            return c

        lax.fori_loop(0, my_ch // 2 - 1, pair_body, 0)

        g0 = my_ch - 2
        wait_gather(g0, rows0, sem0)
        drain_one_out(outb0)
        reduce_chunk(rows0, outb0)
        out_copy(g0, outb0)
        wait_gather(g0 + 1, rows1, sem1)
        drain_one_out(outb1)
        reduce_chunk(rows1, outb1)
        out_copy(g0 + 1, outb1)
        drain_one_out(outb0)
        drain_one_out(outb1)

    return kern(features_bf, pools_chunked)


@jax.jit
def kernel(features, pools):
    features_bf = features.astype(jnp.bfloat16)
    features_u32 = lax.bitcast_convert_type(
        features_bf.reshape(M, DW, 2), jnp.uint32)
    pools_flat = pools.astype(jnp.int32).reshape(-1)
    pad = IDX_ROWS * G_ROWS - pools_flat.shape[0]
    pools_flat = jnp.concatenate(
        [pools_flat, jnp.zeros((pad,), dtype=jnp.int32)]
    )
    pools_chunked = pools_flat.reshape(IDX_ROWS, 1, G_ROWS)
    out = _sc_max_pool(features_u32, pools_chunked)
    out_bf = lax.bitcast_convert_type(out[:M], jnp.bfloat16).reshape(M, D)
    return out_bf.astype(jnp.float32)


# speedup vs baseline: 1.0214x; 1.0214x over previous
import functools

import jax
import jax.numpy as jnp
from jax import lax
from jax.experimental import pallas as pl
from jax.experimental.pallas import tpu as pltpu
from jax.experimental.pallas import tpu_sc as plsc

M = 50000
D = 256
DW = D // 2
K = 16
L = 16

NC, NS = 2, 16
NW = NC * NS
C_OUT = 8
G_ROWS = C_OUT * K

CH0 = 260
CH1 = 132
TOT_CHUNKS = NS * (CH0 + CH1)
M_PAD = TOT_CHUNKS * C_OUT
IDX_ROWS = NS * CH0 + (NS - 1) * CH1 + CH0


def _sc_max_pool(features_bf, pools_chunked):
    mesh = plsc.VectorSubcoreMesh(core_axis_name="c", subcore_axis_name="s")

    @functools.partial(
        pl.kernel,
        mesh=mesh,
        out_type=jax.ShapeDtypeStruct((M_PAD, DW), jnp.uint32),
        scratch_types=[
            pltpu.VMEM((CH0, 1, G_ROWS), jnp.int32),
            pltpu.VMEM((G_ROWS, DW), jnp.uint32),
            pltpu.VMEM((G_ROWS, DW), jnp.uint32),
            pltpu.VMEM((C_OUT, DW), jnp.uint32),
            pltpu.VMEM((C_OUT, DW), jnp.uint32),
            pltpu.SemaphoreType.DMA,
            pltpu.SemaphoreType.DMA,
            pltpu.SemaphoreType.DMA,
        ],
    )
    def kern(feat_hbm, idx_hbm, out_hbm, idx_all, rows0, rows1, outb0,
             outb1, sem0, sem1, sem_o):
        cidx = lax.axis_index("c")
        sidx = lax.axis_index("s")
        is0 = cidx == 0
        my_ch = lax.select(is0, jnp.int32(CH0), jnp.int32(CH1))
        row0_of_tile = lax.select(
            is0, sidx * CH0, NS * CH0 + sidx * CH1)
        base_w = row0_of_tile * C_OUT

        pltpu.sync_copy(idx_hbm.at[pl.ds(row0_of_tile, CH0)], idx_all)

        def fire(g, rows, sem):
            pltpu.async_copy(feat_hbm.at[idx_all.at[g, 0]], rows, sem)

        def wait_gather(g, rows, sem):
            pltpu.make_async_copy(
                feat_hbm.at[idx_all.at[g, 0]], rows, sem).wait()

        def reduce_chunk(rows, outb):
            def r_body(r, c):
                row0 = r * K
                for j in range(DW // L):
                    col = j * L
                    bc = lax.bitcast_convert_type
                    x = rows[row0, pl.ds(col, L)]
                    lo = bc(x << 16, jnp.float32)
                    hi = bc(x, jnp.float32)
                    for k in range(1, K):
                        x = rows[row0 + k, pl.ds(col, L)]
                        lo = jnp.maximum(lo, bc(x << 16, jnp.float32))
                        hi = jnp.maximum(hi, bc(x, jnp.float32))
                    outb[r, pl.ds(col, L)] = (
                        (bc(lo, jnp.uint32) >> 16)
                        | (bc(hi, jnp.uint32) & jnp.uint32(0xFFFF0000)))
                return c

            lax.fori_loop(0, C_OUT, r_body, 0)

        def out_copy(g, outb):
            pltpu.async_copy(
                outb, out_hbm.at[pl.ds(base_w + g * C_OUT, C_OUT)], sem_o)

        def drain_one_out(outb):
            pltpu.make_async_copy(
                outb, out_hbm.at[pl.ds(base_w, C_OUT)], sem_o).wait()

        fire(0, rows0, sem0)
        fire(1, rows1, sem1)

        def pair_body(t, c):
            g0 = 2 * t
            wait_gather(g0, rows0, sem0)

            @pl.when(t > 0)
            def _():
                drain_one_out(outb0)

            reduce_chunk(rows0, outb0)
            out_copy(g0, outb0)
            fire(g0 + 2, rows0, sem0)

            wait_gather(g0 + 1, rows1, sem1)

            @pl.when(t > 0)
            def _():
                drain_one_out(outb1)

            reduce_chunk(rows1, outb1)
            out_copy(g0 + 1, outb1)
            fire(g0 + 3, rows1, sem1)
            return c

        lax.fori_loop(0, my_ch // 2 - 1, pair_body, 0)

        g0 = my_ch - 2
        wait_gather(g0, rows0, sem0)
        drain_one_out(outb0)
        reduce_chunk(rows0, outb0)
        out_copy(g0, outb0)
        wait_gather(g0 + 1, rows1, sem1)
        drain_one_out(outb1)
        reduce_chunk(rows1, outb1)
        out_copy(g0 + 1, outb1)
        drain_one_out(outb0)
        drain_one_out(outb1)

    return kern(features_bf, pools_chunked)


@jax.jit
def kernel(features, pools):
    features_bf = features.astype(jnp.bfloat16)
    features_u32 = lax.bitcast_convert_type(
        features_bf.reshape(M, DW, 2), jnp.uint32)
    pools_flat = pools.astype(jnp.int32).reshape(-1)
    pad = IDX_ROWS * G_ROWS - pools_flat.shape[0]
    pools_flat = jnp.concatenate(
        [pools_flat, jnp.zeros((pad,), dtype=jnp.int32)]
    )
    pools_chunked = pools_flat.reshape(IDX_ROWS, 1, G_ROWS)
    out = _sc_max_pool(features_u32, pools_chunked)
    out_bf = lax.bitcast_convert_type(out[:M], jnp.bfloat16).reshape(M, D)
    return out_bf.astype(jnp.float32)


# speedup vs baseline: 1.8827x; 1.8433x over previous
import functools

import jax
import jax.numpy as jnp
from jax import lax
from jax.experimental import pallas as pl
from jax.experimental.pallas import tpu as pltpu
from jax.experimental.pallas import tpu_sc as plsc

M = 50000
D = 256
K = 16
L = 16

NC, NS = 2, 16
NW = NC * NS
C_OUT = 8
G_ROWS = C_OUT * K

CH0 = 250
CH1 = 142
TOT_CHUNKS = NS * (CH0 + CH1)
M_PAD = TOT_CHUNKS * C_OUT
IDX_ROWS = NS * CH0 + (NS - 1) * CH1 + CH0


def _sc_max_pool(features, pools_chunked):
    mesh = plsc.VectorSubcoreMesh(core_axis_name="c", subcore_axis_name="s")

    @functools.partial(
        pl.kernel,
        mesh=mesh,
        out_type=jax.ShapeDtypeStruct((M_PAD, D), jnp.float32),
        scratch_types=[
            pltpu.VMEM((CH0, 1, G_ROWS), jnp.int32),
            pltpu.VMEM((G_ROWS, D), jnp.float32),
            pltpu.VMEM((G_ROWS, D), jnp.float32),
            pltpu.VMEM((C_OUT, D), jnp.float32),
            pltpu.VMEM((C_OUT, D), jnp.float32),
            pltpu.SemaphoreType.DMA,
            pltpu.SemaphoreType.DMA,
            pltpu.SemaphoreType.DMA,
        ],
    )
    def kern(feat_hbm, idx_hbm, out_hbm, idx_all, rows0, rows1, outb0,
             outb1, sem0, sem1, sem_o):
        cidx = lax.axis_index("c")
        sidx = lax.axis_index("s")
        is0 = cidx == 0
        my_ch = lax.select(is0, jnp.int32(CH0), jnp.int32(CH1))
        row0_of_tile = lax.select(
            is0, sidx * CH0, NS * CH0 + sidx * CH1)
        base_w = row0_of_tile * C_OUT

        pltpu.sync_copy(idx_hbm.at[pl.ds(row0_of_tile, CH0)], idx_all)

        def fire(g, rows, sem):
            pltpu.async_copy(feat_hbm.at[idx_all.at[g, 0]], rows, sem)

        def wait_gather(g, rows, sem):
            pltpu.make_async_copy(
                feat_hbm.at[idx_all.at[g, 0]], rows, sem).wait()

        def reduce_chunk(rows, outb):
            def r_body(r, c):
                row0 = r * K
                for j in range(D // L):
                    col = j * L
                    acc = rows[row0, pl.ds(col, L)]
                    for k in range(1, K):
                        acc = jnp.maximum(acc, rows[row0 + k, pl.ds(col, L)])
                    outb[r, pl.ds(col, L)] = acc
                return c

            lax.fori_loop(0, C_OUT, r_body, 0)

        def out_copy(g, outb):
            pltpu.async_copy(
                outb, out_hbm.at[pl.ds(base_w + g * C_OUT, C_OUT)], sem_o)

        def drain_one_out(outb):
            pltpu.make_async_copy(
                outb, out_hbm.at[pl.ds(base_w, C_OUT)], sem_o).wait()

        fire(0, rows0, sem0)
        fire(1, rows1, sem1)

        def pair_body(t, c):
            g0 = 2 * t
            wait_gather(g0, rows0, sem0)

            @pl.when(t > 0)
            def _():
                drain_one_out(outb0)

            reduce_chunk(rows0, outb0)
            out_copy(g0, outb0)
            fire(g0 + 2, rows0, sem0)

            wait_gather(g0 + 1, rows1, sem1)

            @pl.when(t > 0)
            def _():
                drain_one_out(outb1)

            reduce_chunk(rows1, outb1)
            out_copy(g0 + 1, outb1)
            fire(g0 + 3, rows1, sem1)
            return c

        lax.fori_loop(0, my_ch // 2 - 1, pair_body, 0)

        g0 = my_ch - 2
        wait_gather(g0, rows0, sem0)
        drain_one_out(outb0)
        reduce_chunk(rows0, outb0)
        out_copy(g0, outb0)
        wait_gather(g0 + 1, rows1, sem1)
        drain_one_out(outb1)
        reduce_chunk(rows1, outb1)
        out_copy(g0 + 1, outb1)
        drain_one_out(outb0)
        drain_one_out(outb1)

    return kern(features, pools_chunked)


@jax.jit
def kernel(features, pools):
    pools_flat = pools.astype(jnp.int32).reshape(-1)
    pad = IDX_ROWS * G_ROWS - pools_flat.shape[0]
    pools_flat = jnp.concatenate(
        [pools_flat, jnp.zeros((pad,), dtype=jnp.int32)]
    )
    pools_chunked = pools_flat.reshape(IDX_ROWS, 1, G_ROWS)
    out = _sc_max_pool(features, pools_chunked)
    return out[:M]


# speedup vs baseline: 2.2299x; 1.1844x over previous
import functools

import jax
import jax.numpy as jnp
from jax import lax
from jax.experimental import pallas as pl
from jax.experimental.pallas import tpu as pltpu
from jax.experimental.pallas import tpu_sc as plsc

M = 50000
D = 256
K = 16
L = 16

NC, NS = 2, 16
NW = NC * NS
C_OUT = 8
G_ROWS = C_OUT * K

CH0 = 238
CH1 = 154
TOT_CHUNKS = NS * (CH0 + CH1)
REAL_CHUNKS = M // C_OUT
TAIL_OVL = TOT_CHUNKS - REAL_CHUNKS
IDX_ROWS = NS * CH0 + (NS - 2) * CH1 + CH0
IDX_ROWS = max(IDX_ROWS, REAL_CHUNKS - CH1 + CH0)


def _sc_max_pool(features, pools_chunked):
    mesh = plsc.VectorSubcoreMesh(core_axis_name="c", subcore_axis_name="s")

    @functools.partial(
        pl.kernel,
        mesh=mesh,
        out_type=jax.ShapeDtypeStruct((M, D), jnp.float32),
        scratch_types=[
            pltpu.VMEM((CH0, 1, G_ROWS), jnp.int32),
            pltpu.VMEM((G_ROWS, D), jnp.float32),
            pltpu.VMEM((G_ROWS, D), jnp.float32),
            pltpu.VMEM((C_OUT, D), jnp.float32),
            pltpu.VMEM((C_OUT, D), jnp.float32),
            pltpu.SemaphoreType.DMA,
            pltpu.SemaphoreType.DMA,
            pltpu.SemaphoreType.DMA,
        ],
    )
    def kern(feat_hbm, idx_hbm, out_hbm, idx_all, rows0, rows1, outb0,
             outb1, sem0, sem1, sem_o):
        cidx = lax.axis_index("c")
        sidx = lax.axis_index("s")
        is0 = cidx == 0
        my_ch = lax.select(is0, jnp.int32(CH0), jnp.int32(CH1))
        row0_of_tile = lax.select(
            is0, sidx * CH0,
            NS * CH0 + sidx * CH1
            - jnp.where(sidx == NS - 1, TAIL_OVL, 0).astype(jnp.int32))
        base_w = row0_of_tile * C_OUT

        pltpu.sync_copy(idx_hbm.at[pl.ds(row0_of_tile, CH0)], idx_all)

        def fire(g, rows, sem):
            pltpu.async_copy(feat_hbm.at[idx_all.at[g, 0]], rows, sem)

        def wait_gather(g, rows, sem):
            pltpu.make_async_copy(
                feat_hbm.at[idx_all.at[g, 0]], rows, sem).wait()

        def reduce_chunk(rows, outb):
            def r_body(r, c):
                row0 = r * K
                for j in range(D // L):
                    col = j * L
                    acc = rows[row0, pl.ds(col, L)]
                    for k in range(1, K):
                        acc = jnp.maximum(acc, rows[row0 + k, pl.ds(col, L)])
                    outb[r, pl.ds(col, L)] = acc
                return c

            lax.fori_loop(0, C_OUT, r_body, 0)

        def out_copy(g, outb):
            pltpu.async_copy(
                outb, out_hbm.at[pl.ds(base_w + g * C_OUT, C_OUT)], sem_o)

        def drain_one_out(outb):
            pltpu.make_async_copy(
                outb, out_hbm.at[pl.ds(base_w, C_OUT)], sem_o).wait()

        fire(0, rows0, sem0)
        fire(1, rows1, sem1)

        def pair_body(t, c):
            g0 = 2 * t
            wait_gather(g0, rows0, sem0)

            @pl.when(t > 0)
            def _():
                drain_one_out(outb0)

            reduce_chunk(rows0, outb0)
            out_copy(g0, outb0)
            fire(g0 + 2, rows0, sem0)

            wait_gather(g0 + 1, rows1, sem1)

            @pl.when(t > 0)
            def _():
                drain_one_out(outb1)

            reduce_chunk(rows1, outb1)
            out_copy(g0 + 1, outb1)
            fire(g0 + 3, rows1, sem1)
            return c

        lax.fori_loop(0, my_ch // 2 - 1, pair_body, 0)

        g0 = my_ch - 2
        wait_gather(g0, rows0, sem0)
        drain_one_out(outb0)
        reduce_chunk(rows0, outb0)
        out_copy(g0, outb0)
        wait_gather(g0 + 1, rows1, sem1)
        drain_one_out(outb1)
        reduce_chunk(rows1, outb1)
        out_copy(g0 + 1, outb1)
        drain_one_out(outb0)
        drain_one_out(outb1)

    return kern(features, pools_chunked)


@jax.jit
def kernel(features, pools):
    pools_flat = pools.astype(jnp.int32).reshape(-1)
    pad = IDX_ROWS * G_ROWS - pools_flat.shape[0]
    pools_flat = jnp.concatenate(
        [pools_flat, jnp.zeros((pad,), dtype=jnp.int32)]
    )
    pools_chunked = pools_flat.reshape(IDX_ROWS, 1, G_ROWS)
    return _sc_max_pool(features, pools_chunked)


# speedup vs baseline: 2.2506x; 1.0093x over previous
import functools

import jax
import jax.numpy as jnp
from jax import lax
from jax.experimental import pallas as pl
from jax.experimental.pallas import tpu as pltpu
from jax.experimental.pallas import tpu_sc as plsc

M = 50000
D = 256
K = 16
L = 16

NC, NS = 2, 16
NW = NC * NS
C_OUT = 8
G_ROWS = C_OUT * K

CH0 = 238
CH1 = 154
TOT_CHUNKS = NS * (CH0 + CH1)
REAL_CHUNKS = M // C_OUT
TAIL_OVL = TOT_CHUNKS - REAL_CHUNKS
IDX_ROWS = REAL_CHUNKS


def _sc_max_pool(features, pools_chunked):
    mesh = plsc.VectorSubcoreMesh(core_axis_name="c", subcore_axis_name="s")

    @functools.partial(
        pl.kernel,
        mesh=mesh,
        out_type=jax.ShapeDtypeStruct((M, D), jnp.float32),
        scratch_types=[
            pltpu.VMEM((CH0, 1, G_ROWS), jnp.int32),
            pltpu.VMEM((G_ROWS, D), jnp.float32),
            pltpu.VMEM((G_ROWS, D), jnp.float32),
            pltpu.VMEM((C_OUT, D), jnp.float32),
            pltpu.VMEM((C_OUT, D), jnp.float32),
            pltpu.SemaphoreType.DMA,
            pltpu.SemaphoreType.DMA,
            pltpu.SemaphoreType.DMA,
        ],
    )
    def kern(feat_hbm, idx_hbm, out_hbm, idx_all, rows0, rows1, outb0,
             outb1, sem0, sem1, sem_o):
        cidx = lax.axis_index("c")
        sidx = lax.axis_index("s")
        is0 = cidx == 0
        my_ch = lax.select(is0, jnp.int32(CH0), jnp.int32(CH1))
        row0_of_tile = lax.select(
            is0, sidx * CH0,
            NS * CH0 + sidx * CH1
            - jnp.where(sidx == NS - 1, TAIL_OVL, 0).astype(jnp.int32))
        base_w = row0_of_tile * C_OUT

        @pl.when(is0)
        def _():
            pltpu.sync_copy(idx_hbm.at[pl.ds(row0_of_tile, CH0)], idx_all)

        @pl.when(jnp.logical_not(is0))
        def _():
            pltpu.sync_copy(idx_hbm.at[pl.ds(row0_of_tile, CH1)],
                            idx_all.at[pl.ds(0, CH1)])

        def fire(g, rows, sem):
            pltpu.async_copy(feat_hbm.at[idx_all.at[g, 0]], rows, sem)

        def wait_gather(g, rows, sem):
            pltpu.make_async_copy(
                feat_hbm.at[idx_all.at[g, 0]], rows, sem).wait()

        def reduce_chunk(rows, outb):
            def r_body(r, c):
                row0 = r * K
                for j in range(D // L):
                    col = j * L
                    acc = rows[row0, pl.ds(col, L)]
                    for k in range(1, K):
                        acc = jnp.maximum(acc, rows[row0 + k, pl.ds(col, L)])
                    outb[r, pl.ds(col, L)] = acc
                return c

            lax.fori_loop(0, C_OUT, r_body, 0)

        def out_copy(g, outb):
            pltpu.async_copy(
                outb, out_hbm.at[pl.ds(base_w + g * C_OUT, C_OUT)], sem_o)

        def drain_one_out(outb):
            pltpu.make_async_copy(
                outb, out_hbm.at[pl.ds(base_w, C_OUT)], sem_o).wait()

        fire(0, rows0, sem0)
        fire(1, rows1, sem1)

        def pair_body(t, c):
            g0 = 2 * t
            wait_gather(g0, rows0, sem0)

            @pl.when(t > 0)
            def _():
                drain_one_out(outb0)

            reduce_chunk(rows0, outb0)
            out_copy(g0, outb0)
            fire(g0 + 2, rows0, sem0)

            wait_gather(g0 + 1, rows1, sem1)

            @pl.when(t > 0)
            def _():
                drain_one_out(outb1)

            reduce_chunk(rows1, outb1)
            out_copy(g0 + 1, outb1)
            fire(g0 + 3, rows1, sem1)
            return c

        lax.fori_loop(0, my_ch // 2 - 1, pair_body, 0)

        g0 = my_ch - 2
        wait_gather(g0, rows0, sem0)
        drain_one_out(outb0)
        reduce_chunk(rows0, outb0)
        out_copy(g0, outb0)
        wait_gather(g0 + 1, rows1, sem1)
        drain_one_out(outb1)
        reduce_chunk(rows1, outb1)
        out_copy(g0 + 1, outb1)
        drain_one_out(outb0)
        drain_one_out(outb1)

    return kern(features, pools_chunked)


@jax.jit
def kernel(features, pools):
    pools_chunked = pools.astype(jnp.int32).reshape(IDX_ROWS, 1, G_ROWS)
    return _sc_max_pool(features, pools_chunked)


# speedup vs baseline: 2.6417x; 1.1737x over previous
import functools

import jax
import jax.numpy as jnp
from jax import lax
from jax.experimental import pallas as pl
from jax.experimental.pallas import tpu as pltpu
from jax.experimental.pallas import tpu_sc as plsc

M = 50000
D = 256
K = 16
L = 16

NC, NS = 2, 16
NW = NC * NS
C_OUT = 8
G_ROWS = C_OUT * K

CH0 = 196
CH1 = 196
TOT_CHUNKS = NS * (CH0 + CH1)
REAL_CHUNKS = M // C_OUT
TAIL_OVL = TOT_CHUNKS - REAL_CHUNKS
IDX_ROWS = REAL_CHUNKS


def _sc_max_pool(features, pools_chunked):
    mesh = plsc.VectorSubcoreMesh(core_axis_name="c", subcore_axis_name="s")

    @functools.partial(
        pl.kernel,
        mesh=mesh,
        out_type=jax.ShapeDtypeStruct((M, D), jnp.float32),
        scratch_types=[
            pltpu.VMEM((CH0, 1, G_ROWS), jnp.int32),
            pltpu.VMEM((G_ROWS, D), jnp.float32),
            pltpu.VMEM((G_ROWS, D), jnp.float32),
            pltpu.VMEM((C_OUT, D), jnp.float32),
            pltpu.VMEM((C_OUT, D), jnp.float32),
            pltpu.SemaphoreType.DMA,
            pltpu.SemaphoreType.DMA,
            pltpu.SemaphoreType.DMA,
        ],
    )
    def kern(feat_hbm, idx_hbm, out_hbm, idx_all, rows0, rows1, outb0,
             outb1, sem0, sem1, sem_o):
        cidx = lax.axis_index("c")
        sidx = lax.axis_index("s")
        is0 = cidx == 0
        my_ch = lax.select(is0, jnp.int32(CH0), jnp.int32(CH1))
        row0_of_tile = lax.select(
            is0, sidx * CH0,
            NS * CH0 + sidx * CH1
            - jnp.where(sidx == NS - 1, TAIL_OVL, 0).astype(jnp.int32))
        base_w = row0_of_tile * C_OUT

        @pl.when(is0)
        def _():
            pltpu.sync_copy(idx_hbm.at[pl.ds(row0_of_tile, CH0)], idx_all)

        @pl.when(jnp.logical_not(is0))
        def _():
            pltpu.sync_copy(idx_hbm.at[pl.ds(row0_of_tile, CH1)],
                            idx_all.at[pl.ds(0, CH1)])

        def fire(g, rows, sem):
            pltpu.async_copy(feat_hbm.at[idx_all.at[g, 0]], rows, sem)

        def wait_gather(g, rows, sem):
            pltpu.make_async_copy(
                feat_hbm.at[idx_all.at[g, 0]], rows, sem).wait()

        def reduce_chunk(rows, outb):
            def r_body(r, c):
                row0 = r * K
                for j in range(D // L):
                    col = j * L
                    acc = rows[row0, pl.ds(col, L)]
                    for k in range(1, K):
                        acc = jnp.maximum(acc, rows[row0 + k, pl.ds(col, L)])
                    outb[r, pl.ds(col, L)] = acc
                return c

            lax.fori_loop(0, C_OUT, r_body, 0)

        def out_copy(g, outb):
            pltpu.async_copy(
                outb, out_hbm.at[pl.ds(base_w + g * C_OUT, C_OUT)], sem_o)

        def drain_one_out(outb):
            pltpu.make_async_copy(
                outb, out_hbm.at[pl.ds(base_w, C_OUT)], sem_o).wait()

        fire(0, rows0, sem0)
        fire(1, rows1, sem1)

        def pair_body(t, c):
            g0 = 2 * t
            wait_gather(g0, rows0, sem0)

            @pl.when(t > 0)
            def _():
                drain_one_out(outb0)

            reduce_chunk(rows0, outb0)
            out_copy(g0, outb0)
            fire(g0 + 2, rows0, sem0)

            wait_gather(g0 + 1, rows1, sem1)

            @pl.when(t > 0)
            def _():
                drain_one_out(outb1)

            reduce_chunk(rows1, outb1)
            out_copy(g0 + 1, outb1)
            fire(g0 + 3, rows1, sem1)
            return c

        lax.fori_loop(0, my_ch // 2 - 1, pair_body, 0)

        g0 = my_ch - 2
        wait_gather(g0, rows0, sem0)
        drain_one_out(outb0)
        reduce_chunk(rows0, outb0)
        out_copy(g0, outb0)
        wait_gather(g0 + 1, rows1, sem1)
        drain_one_out(outb1)
        reduce_chunk(rows1, outb1)
        out_copy(g0 + 1, outb1)
        drain_one_out(outb0)
        drain_one_out(outb1)

    return kern(features, pools_chunked)


@jax.jit
def kernel(features, pools):
    pools_chunked = pools.astype(jnp.int32).reshape(IDX_ROWS, 1, G_ROWS)
    return _sc_max_pool(features, pools_chunked)


# speedup vs baseline: 2.8339x; 1.0728x over previous
import functools

import jax
import jax.numpy as jnp
from jax import lax
from jax.experimental import pallas as pl
from jax.experimental.pallas import tpu as pltpu
from jax.experimental.pallas import tpu_sc as plsc

M = 50000
D = 256
K = 16
L = 16

NC, NS = 2, 16
NW = NC * NS
C_OUT = 8
G_ROWS = C_OUT * K

CH0 = 198
CH1 = 198
TOT_CHUNKS = NS * (CH0 + CH1)
REAL_CHUNKS = M // C_OUT
TAIL_OVL = TOT_CHUNKS - REAL_CHUNKS
IDX_ROWS = REAL_CHUNKS


def _sc_max_pool(features, pools_chunked):
    mesh = plsc.VectorSubcoreMesh(core_axis_name="c", subcore_axis_name="s")

    @functools.partial(
        pl.kernel,
        mesh=mesh,
        out_type=jax.ShapeDtypeStruct((M, D), jnp.float32),
        scratch_types=[
            pltpu.VMEM((CH0, 1, G_ROWS), jnp.int32),
            pltpu.VMEM((G_ROWS, D), jnp.float32),
            pltpu.VMEM((G_ROWS, D), jnp.float32),
            pltpu.VMEM((G_ROWS, D), jnp.float32),
            pltpu.VMEM((C_OUT, D), jnp.float32),
            pltpu.VMEM((C_OUT, D), jnp.float32),
            pltpu.VMEM((C_OUT, D), jnp.float32),
            pltpu.SemaphoreType.DMA,
            pltpu.SemaphoreType.DMA,
            pltpu.SemaphoreType.DMA,
            pltpu.SemaphoreType.DMA,
        ],
    )
    def kern(feat_hbm, idx_hbm, out_hbm, idx_all, rows0, rows1, rows2,
             outb0, outb1, outb2, sem0, sem1, sem2, sem_o):
        cidx = lax.axis_index("c")
        sidx = lax.axis_index("s")
        is0 = cidx == 0
        my_ch = lax.select(is0, jnp.int32(CH0), jnp.int32(CH1))
        row0_of_tile = lax.select(
            is0, sidx * CH0,
            NS * CH0 + sidx * CH1
            - jnp.where(sidx == NS - 1, TAIL_OVL, 0).astype(jnp.int32))
        base_w = row0_of_tile * C_OUT

        @pl.when(is0)
        def _():
            pltpu.sync_copy(idx_hbm.at[pl.ds(row0_of_tile, CH0)], idx_all)

        @pl.when(jnp.logical_not(is0))
        def _():
            pltpu.sync_copy(idx_hbm.at[pl.ds(row0_of_tile, CH1)],
                            idx_all.at[pl.ds(0, CH1)])

        def fire(g, rows, sem):
            pltpu.async_copy(feat_hbm.at[idx_all.at[g, 0]], rows, sem)

        def wait_gather(g, rows, sem):
            pltpu.make_async_copy(
                feat_hbm.at[idx_all.at[g, 0]], rows, sem).wait()

        def reduce_chunk(rows, outb):
            def r_body(r, c):
                row0 = r * K
                for j in range(D // L):
                    col = j * L
                    acc = rows[row0, pl.ds(col, L)]
                    for k in range(1, K):
                        acc = jnp.maximum(acc, rows[row0 + k, pl.ds(col, L)])
                    outb[r, pl.ds(col, L)] = acc
                return c

            lax.fori_loop(0, C_OUT, r_body, 0)

        def out_copy(g, outb):
            pltpu.async_copy(
                outb, out_hbm.at[pl.ds(base_w + g * C_OUT, C_OUT)], sem_o)

        def drain_one_out(outb):
            pltpu.make_async_copy(
                outb, out_hbm.at[pl.ds(base_w, C_OUT)], sem_o).wait()

        fire(0, rows0, sem0)
        fire(1, rows1, sem1)
        fire(2, rows2, sem2)

        bufs = ((rows0, outb0, sem0), (rows1, outb1, sem1),
                (rows2, outb2, sem2))

        def tri_body(t, c):
            g0 = 3 * t
            for i, (rows, outb, sem) in enumerate(bufs):
                wait_gather(g0 + i, rows, sem)

                @pl.when(t > 0)
                def _():
                    drain_one_out(outb)

                reduce_chunk(rows, outb)
                out_copy(g0 + i, outb)
                fire(g0 + i + 3, rows, sem)
            return c

        lax.fori_loop(0, my_ch // 3 - 1, tri_body, 0)

        g0 = my_ch - 3
        for i, (rows, outb, sem) in enumerate(bufs):
            wait_gather(g0 + i, rows, sem)
            drain_one_out(outb)
            reduce_chunk(rows, outb)
            out_copy(g0 + i, outb)
        for _, outb, _sem in bufs:
            drain_one_out(outb)

    return kern(features, pools_chunked)


@jax.jit
def kernel(features, pools):
    pools_chunked = pools.astype(jnp.int32).reshape(IDX_ROWS, 1, G_ROWS)
    return _sc_max_pool(features, pools_chunked)
